# baseline (speedup 1.0000x reference)
import jax
import jax.numpy as jnp
from jax import lax
from jax.experimental import pallas as pl
from jax.experimental.pallas import tpu as pltpu
from jax.experimental.pallas import tpu_sc as plsc

_N_ELEMS = 6400000
_NSEG = 100000
_NC, _NS = 2, 16
_NW = _NC * _NS
_NSEG_PAD = 100096
_PER_TILE = _N_ELEMS // _NW
_B = 2000
_NCHUNK = _PER_TILE // _B
_NPAIR = _NCHUNK // 2
_VREGS = _B // 16
_UNROLL = 10
_FLUSH = _NSEG_PAD // _NS


def _sc_body(ea_hbm, eza_hbm, n_hbm, out_hbm, part_hbm,
             ea0, ea1, ez0, ez1, n0, n1, t0, t1, nb, accl, accs,
             isem0, isem1, osem0, osem1):
    c = lax.axis_index("c")
    s = lax.axis_index("s")
    wid = c * _NS + s
    base_el = wid * _PER_TILE

    ea = (ea0, ea1)
    ez = (ez0, ez1)
    nn = (n0, n1)
    tt = (t0, t1)
    isem = (isem0, isem1)
    osem = (osem0, osem1)

    zero16 = jnp.zeros((16,), jnp.float32)

    def z16(i, _):
        for u in range(16):
            accl[pl.ds((i * 16 + u) * 16, 16)] = zero16
        return 0
    lax.fori_loop(0, _NSEG_PAD // 256, z16, 0)

    pltpu.sync_copy(accl.at[pl.ds(0, _FLUSH)],
                    accs.at[pl.ds(s * _FLUSH, _FLUSH)])
    plsc.subcore_barrier()

    def start_in(slot, k):
        off = base_el + k * _B
        pltpu.async_copy(ea_hbm.at[pl.ds(off, _B)], ea[slot], isem[slot])
        pltpu.async_copy(eza_hbm.at[pl.ds(off, _B)], ez[slot], isem[slot])
        pltpu.async_copy(n_hbm.at[pl.ds(off, _B)], nn[slot], isem[slot])

    def wait_in(slot, k):
        off = base_el + k * _B
        pltpu.make_async_copy(ea_hbm.at[pl.ds(off, _B)], ea[slot], isem[slot]).wait()
        pltpu.make_async_copy(eza_hbm.at[pl.ds(off, _B)], ez[slot], isem[slot]).wait()
        pltpu.make_async_copy(n_hbm.at[pl.ds(off, _B)], nn[slot], isem[slot]).wait()

    def wait_out(slot, k):
        off = base_el + k * _B
        pltpu.make_async_copy(tt[slot], out_hbm.at[pl.ds(off, _B)], osem[slot]).wait()

    iota16 = lax.iota(jnp.int32, 16)
    stride_idx = iota16 * _VREGS

    def compute(slot, k):
        @plsc.parallel_loop(0, _VREGS, unroll=_UNROLL)
        def _(i):
            sl = pl.ds(i * 16, 16)
            tt[slot][sl] = ea[slot][sl] + ez[slot][sl]
        pltpu.async_copy(tt[slot], out_hbm.at[pl.ds(base_el + k * _B, _B)],
                         osem[slot])

        @plsc.parallel_loop(0, _VREGS, unroll=_UNROLL)
        def _(i):
            idx = stride_idx + i
            t = plsc.load_gather(tt[slot], [idx])
            n16 = plsc.load_gather(nn[slot], [idx])
            plsc.addupdate_scatter(accl, [n16], t)

    start_in(0, 0)
    start_in(1, 1)
    wait_in(0, 0)
    compute(0, 0)
    start_in(0, 2)
    wait_in(1, 1)
    compute(1, 1)
    start_in(1, 3)

    def pair(k2, _):
        k = 2 * k2
        wait_out(0, k - 2)
        wait_in(0, k)
        compute(0, k)

        @pl.when(k2 < _NPAIR - 1)
        def _():
            start_in(0, k + 2)
        wait_out(1, k - 1)
        wait_in(1, k + 1)
        compute(1, k + 1)

        @pl.when(k2 < _NPAIR - 1)
        def _():
            start_in(1, k + 3)
        return 0
    lax.fori_loop(1, _NPAIR, pair, 0)
    wait_out(0, _NCHUNK - 2)
    wait_out(1, _NCHUNK - 1)

    pltpu.sync_copy(n_hbm.at[pl.ds(base_el, 16)], nb)
    lo = nb[pl.ds(0, 16)][0]
    pltpu.sync_copy(n_hbm.at[pl.ds(base_el + _PER_TILE - 16, 16)], nb)
    hi = nb[pl.ds(0, 16)][15]

    fbase = (lo // 256) * 256
    nbat = (hi + 1 - fbase + 255) // 256

    def fl(b, _):
        off_b = fbase + b * 256
        for u in range(16):
            off = off_b + u * 16
            pltpu.async_copy(accl.at[pl.ds(off, 16)], accs.at[off + iota16],
                             isem0, add=True)
        for u in range(16):
            off = off_b + u * 16
            pltpu.make_async_copy(accl.at[pl.ds(off, 16)],
                                  accs.at[off + iota16], isem0).wait()
        return 0
    lax.fori_loop(0, nbat, fl, 0)

    plsc.subcore_barrier()
    off = s * _FLUSH
    pltpu.sync_copy(accs.at[pl.ds(off, _FLUSH)], accl.at[pl.ds(0, _FLUSH)])
    pltpu.sync_copy(accl.at[pl.ds(0, _FLUSH)],
                    part_hbm.at[pl.ds(c * _NSEG_PAD + off, _FLUSH)])


_sc_call = pl.kernel(
    _sc_body,
    out_type=(
        jax.ShapeDtypeStruct((_N_ELEMS,), jnp.float32),
        jax.ShapeDtypeStruct((_NC * _NSEG_PAD,), jnp.float32),
    ),
    mesh=plsc.VectorSubcoreMesh(core_axis_name="c", subcore_axis_name="s",
                                num_cores=_NC, num_subcores=_NS),
    compiler_params=pltpu.CompilerParams(needs_layout_passes=False),
    scratch_types=[
        pltpu.VMEM((_B,), jnp.float32),
        pltpu.VMEM((_B,), jnp.float32),
        pltpu.VMEM((_B,), jnp.float32),
        pltpu.VMEM((_B,), jnp.float32),
        pltpu.VMEM((_B,), jnp.int32),
        pltpu.VMEM((_B,), jnp.int32),
        pltpu.VMEM((_B,), jnp.float32),
        pltpu.VMEM((_B,), jnp.float32),
        pltpu.VMEM((16,), jnp.int32),
        pltpu.VMEM((_NSEG_PAD,), jnp.float32),
        pltpu.VMEM_SHARED((_NSEG_PAD,), jnp.float32),
        pltpu.SemaphoreType.DMA,
        pltpu.SemaphoreType.DMA,
        pltpu.SemaphoreType.DMA,
        pltpu.SemaphoreType.DMA,
    ],
)


def _combine_body(p_ref, o_ref):
    o_ref[...] = p_ref[0] + p_ref[1]


def kernel(Ea, Eza, N):
    total, part = _sc_call(Ea, Eza, N)
    comb = pl.pallas_call(
        _combine_body,
        out_shape=jax.ShapeDtypeStruct((_NSEG_PAD // 128, 128), jnp.float32),
    )(part.reshape(_NC, _NSEG_PAD // 128, 128))
    E = comb.reshape(-1)[:_NSEG]
    return (total, E)


# speedup vs baseline: 1.0277x; 1.0277x over previous
import jax
import jax.numpy as jnp
from jax import lax
from jax.experimental import pallas as pl
from jax.experimental.pallas import tpu as pltpu
from jax.experimental.pallas import tpu_sc as plsc

_N_ELEMS = 6400000
_NSEG = 100000
_NC, _NS = 2, 16
_NW = _NC * _NS
_NSEG_PAD = 100096
_PER_TILE = _N_ELEMS // _NW
_B = 2000
_NCHUNK = _PER_TILE // _B
_NPAIR = _NCHUNK // 2
_VREGS = _B // 16
_UNROLL = 5
_FLUSH = _NSEG_PAD // _NS


def _sc_body(ea_hbm, eza_hbm, n_hbm, out_hbm, part_hbm,
             ea0, ea1, ez0, ez1, n0, n1, t0, t1, nb, accl, accs,
             isem0, isem1, osem0, osem1):
    c = lax.axis_index("c")
    s = lax.axis_index("s")
    wid = c * _NS + s
    base_el = wid * _PER_TILE

    ea = (ea0, ea1)
    ez = (ez0, ez1)
    nn = (n0, n1)
    tt = (t0, t1)
    isem = (isem0, isem1)
    osem = (osem0, osem1)

    zero16 = jnp.zeros((16,), jnp.float32)

    def z16(i, _):
        for u in range(16):
            accl[pl.ds((i * 16 + u) * 16, 16)] = zero16
        return 0
    lax.fori_loop(0, _NSEG_PAD // 256, z16, 0)

    pltpu.sync_copy(accl.at[pl.ds(0, _FLUSH)],
                    accs.at[pl.ds(s * _FLUSH, _FLUSH)])
    plsc.subcore_barrier()

    def start_in(slot, k):
        off = base_el + k * _B
        pltpu.async_copy(ea_hbm.at[pl.ds(off, _B)], ea[slot], isem[slot])
        pltpu.async_copy(eza_hbm.at[pl.ds(off, _B)], ez[slot], isem[slot])
        pltpu.async_copy(n_hbm.at[pl.ds(off, _B)], nn[slot], isem[slot])

    def wait_in(slot, k):
        off = base_el + k * _B
        pltpu.make_async_copy(ea_hbm.at[pl.ds(off, _B)], ea[slot], isem[slot]).wait()
        pltpu.make_async_copy(eza_hbm.at[pl.ds(off, _B)], ez[slot], isem[slot]).wait()
        pltpu.make_async_copy(n_hbm.at[pl.ds(off, _B)], nn[slot], isem[slot]).wait()

    def wait_out(slot, k):
        off = base_el + k * _B
        pltpu.make_async_copy(tt[slot], out_hbm.at[pl.ds(off, _B)], osem[slot]).wait()

    iota16 = lax.iota(jnp.int32, 16)
    stride_idx = iota16 * _VREGS

    def compute(slot, k):
        @plsc.parallel_loop(0, _VREGS, unroll=_UNROLL)
        def _(i):
            sl = pl.ds(i * 16, 16)
            tt[slot][sl] = ea[slot][sl] + ez[slot][sl]
        pltpu.async_copy(tt[slot], out_hbm.at[pl.ds(base_el + k * _B, _B)],
                         osem[slot])

        @plsc.parallel_loop(0, _VREGS, unroll=_UNROLL)
        def _(i):
            idx = stride_idx + i
            t = plsc.load_gather(tt[slot], [idx])
            n16 = plsc.load_gather(nn[slot], [idx])
            plsc.addupdate_scatter(accl, [n16], t)

    start_in(0, 0)
    start_in(1, 1)
    wait_in(0, 0)
    compute(0, 0)
    start_in(0, 2)
    wait_in(1, 1)
    compute(1, 1)
    start_in(1, 3)

    def pair(k2, _):
        k = 2 * k2
        wait_out(0, k - 2)
        wait_in(0, k)
        compute(0, k)

        @pl.when(k2 < _NPAIR - 1)
        def _():
            start_in(0, k + 2)
        wait_out(1, k - 1)
        wait_in(1, k + 1)
        compute(1, k + 1)

        @pl.when(k2 < _NPAIR - 1)
        def _():
            start_in(1, k + 3)
        return 0
    lax.fori_loop(1, _NPAIR, pair, 0)
    wait_out(0, _NCHUNK - 2)
    wait_out(1, _NCHUNK - 1)

    pltpu.sync_copy(n_hbm.at[pl.ds(base_el, 16)], nb)
    lo = nb[pl.ds(0, 16)][0]
    pltpu.sync_copy(n_hbm.at[pl.ds(base_el + _PER_TILE - 16, 16)], nb)
    hi = nb[pl.ds(0, 16)][15]

    fbase = (lo // 256) * 256
    nbat = (hi + 1 - fbase + 255) // 256

    def fl(b, _):
        off_b = fbase + b * 256
        for u in range(16):
            off = off_b + u * 16
            pltpu.async_copy(accl.at[pl.ds(off, 16)], accs.at[off + iota16],
                             isem0, add=True)
        for u in range(16):
            off = off_b + u * 16
            pltpu.make_async_copy(accl.at[pl.ds(off, 16)],
                                  accs.at[off + iota16], isem0).wait()
        return 0
    lax.fori_loop(0, nbat, fl, 0)

    plsc.subcore_barrier()
    off = s * _FLUSH
    pltpu.sync_copy(accs.at[pl.ds(off, _FLUSH)], accl.at[pl.ds(0, _FLUSH)])
    pltpu.sync_copy(accl.at[pl.ds(0, _FLUSH)],
                    part_hbm.at[pl.ds(c * _NSEG_PAD + off, _FLUSH)])


_sc_call = pl.kernel(
    _sc_body,
    out_type=(
        jax.ShapeDtypeStruct((_N_ELEMS,), jnp.float32),
        jax.ShapeDtypeStruct((_NC * _NSEG_PAD,), jnp.float32),
    ),
    mesh=plsc.VectorSubcoreMesh(core_axis_name="c", subcore_axis_name="s",
                                num_cores=_NC, num_subcores=_NS),
    compiler_params=pltpu.CompilerParams(needs_layout_passes=False),
    scratch_types=[
        pltpu.VMEM((_B,), jnp.float32),
        pltpu.VMEM((_B,), jnp.float32),
        pltpu.VMEM((_B,), jnp.float32),
        pltpu.VMEM((_B,), jnp.float32),
        pltpu.VMEM((_B,), jnp.int32),
        pltpu.VMEM((_B,), jnp.int32),
        pltpu.VMEM((_B,), jnp.float32),
        pltpu.VMEM((_B,), jnp.float32),
        pltpu.VMEM((16,), jnp.int32),
        pltpu.VMEM((_NSEG_PAD,), jnp.float32),
        pltpu.VMEM_SHARED((_NSEG_PAD,), jnp.float32),
        pltpu.SemaphoreType.DMA,
        pltpu.SemaphoreType.DMA,
        pltpu.SemaphoreType.DMA,
        pltpu.SemaphoreType.DMA,
    ],
)


def _combine_body(p_ref, o_ref):
    o_ref[...] = p_ref[0] + p_ref[1]


def kernel(Ea, Eza, N):
    total, part = _sc_call(Ea, Eza, N)
    comb = pl.pallas_call(
        _combine_body,
        out_shape=jax.ShapeDtypeStruct((_NSEG_PAD // 128, 128), jnp.float32),
    )(part.reshape(_NC, _NSEG_PAD // 128, 128))
    E = comb.reshape(-1)[:_NSEG]
    return (total, E)


# speedup vs baseline: 1.0295x; 1.0017x over previous
import jax
import jax.numpy as jnp
from jax import lax
from jax.experimental import pallas as pl
from jax.experimental.pallas import tpu as pltpu
from jax.experimental.pallas import tpu_sc as plsc

_N_ELEMS = 6400000
_NSEG = 100000
_NC, _NS = 2, 16
_NW = _NC * _NS
_NSEG_PAD = 100096
_PER_TILE = _N_ELEMS // _NW
_B = 2000
_NCHUNK = _PER_TILE // _B
_NTRI = (_NCHUNK - 1) // 3
_VREGS = _B // 16
_UNROLL = 5
_FLUSH = _NSEG_PAD // _NS


def _sc_body(ea_hbm, eza_hbm, n_hbm, out_hbm, part_hbm,
             ea0, ea1, ea2, ez0, ez1, ez2, n0, n1, n2, t0, t1, t2,
             nb, accl, accs,
             isem0, isem1, isem2, osem0, osem1, osem2, fsem):
    c = lax.axis_index("c")
    s = lax.axis_index("s")
    wid = c * _NS + s
    base_el = wid * _PER_TILE

    ea = (ea0, ea1, ea2)
    ez = (ez0, ez1, ez2)
    nn = (n0, n1, n2)
    tt = (t0, t1, t2)
    isem = (isem0, isem1, isem2)
    osem = (osem0, osem1, osem2)

    zero16 = jnp.zeros((16,), jnp.float32)

    def z16(i, _):
        for u in range(16):
            accl[pl.ds((i * 16 + u) * 16, 16)] = zero16
        return 0
    lax.fori_loop(0, _NSEG_PAD // 256, z16, 0)

    pltpu.sync_copy(accl.at[pl.ds(0, _FLUSH)],
                    accs.at[pl.ds(s * _FLUSH, _FLUSH)])
    plsc.subcore_barrier()

    def start_in(slot, k):
        off = base_el + k * _B
        pltpu.async_copy(ea_hbm.at[pl.ds(off, _B)], ea[slot], isem[slot])
        pltpu.async_copy(eza_hbm.at[pl.ds(off, _B)], ez[slot], isem[slot])
        pltpu.async_copy(n_hbm.at[pl.ds(off, _B)], nn[slot], isem[slot])

    def wait_in(slot, k):
        off = base_el + k * _B
        pltpu.make_async_copy(ea_hbm.at[pl.ds(off, _B)], ea[slot], isem[slot]).wait()
        pltpu.make_async_copy(eza_hbm.at[pl.ds(off, _B)], ez[slot], isem[slot]).wait()
        pltpu.make_async_copy(n_hbm.at[pl.ds(off, _B)], nn[slot], isem[slot]).wait()

    def wait_out(slot, k):
        off = base_el + k * _B
        pltpu.make_async_copy(tt[slot], out_hbm.at[pl.ds(off, _B)], osem[slot]).wait()

    def issue_out(slot, k):
        pltpu.async_copy(tt[slot], out_hbm.at[pl.ds(base_el + k * _B, _B)],
                         osem[slot])

    iota16 = lax.iota(jnp.int32, 16)
    stride_idx = iota16 * _VREGS

    def add_one(slot, i):
        sl = pl.ds(i * 16, 16)
        tt[slot][sl] = ea[slot][sl] + ez[slot][sl]

    def acc_one(slot, i):
        idx = stride_idx + i
        t = plsc.load_gather(tt[slot], [idx])
        n16 = plsc.load_gather(nn[slot], [idx])
        plsc.addupdate_scatter(accl, [n16], t)

    def add_loop(slot):
        @plsc.parallel_loop(0, _VREGS, unroll=_UNROLL)
        def _(i):
            add_one(slot, i)

    def fused_loop(slot, prev):
        @plsc.parallel_loop(0, _VREGS, unroll=_UNROLL)
        def _(i):
            add_one(slot, i)
            acc_one(prev, i)

    def acc_loop(slot):
        @plsc.parallel_loop(0, _VREGS, unroll=_UNROLL)
        def _(i):
            acc_one(slot, i)

    start_in(0, 0)
    start_in(1, 1)
    start_in(2, 2)
    wait_in(0, 0)
    add_loop(0)
    issue_out(0, 0)

    def phase(k, slot, prev):
        wait_in(slot, k)

        @pl.when(k >= 3)
        def _():
            wait_out(slot, k - 3)
        fused_loop(slot, prev)
        issue_out(slot, k)

        @pl.when(k <= _NCHUNK - 3)
        def _():
            start_in(prev, k + 2)

    def tri(t3, _):
        k = 3 * t3 + 1
        phase(k, 1, 0)
        phase(k + 1, 2, 1)
        phase(k + 2, 0, 2)
        return 0
    lax.fori_loop(0, _NTRI, tri, 0)

    acc_loop((_NCHUNK - 1) % 3)
    wait_out(1, _NCHUNK - 3)
    wait_out(2, _NCHUNK - 2)
    wait_out(0, _NCHUNK - 1)

    pltpu.sync_copy(n_hbm.at[pl.ds(base_el, 16)], nb)
    lo = nb[pl.ds(0, 16)][0]
    pltpu.sync_copy(n_hbm.at[pl.ds(base_el + _PER_TILE - 16, 16)], nb)
    hi = nb[pl.ds(0, 16)][15]

    fbase = (lo // 256) * 256
    nbat = (hi + 1 - fbase + 255) // 256

    def fl(b, _):
        off_b = fbase + b * 256
        for u in range(16):
            off = off_b + u * 16
            pltpu.async_copy(accl.at[pl.ds(off, 16)], accs.at[off + iota16],
                             fsem, add=True)
        for u in range(16):
            off = off_b + u * 16
            pltpu.make_async_copy(accl.at[pl.ds(off, 16)],
                                  accs.at[off + iota16], fsem).wait()
        return 0
    lax.fori_loop(0, nbat, fl, 0)

    plsc.subcore_barrier()
    off = s * _FLUSH
    pltpu.sync_copy(accs.at[pl.ds(off, _FLUSH)], accl.at[pl.ds(0, _FLUSH)])
    pltpu.sync_copy(accl.at[pl.ds(0, _FLUSH)],
                    part_hbm.at[pl.ds(c * _NSEG_PAD + off, _FLUSH)])


_sc_call = pl.kernel(
    _sc_body,
    out_type=(
        jax.ShapeDtypeStruct((_N_ELEMS,), jnp.float32),
        jax.ShapeDtypeStruct((_NC * _NSEG_PAD,), jnp.float32),
    ),
    mesh=plsc.VectorSubcoreMesh(core_axis_name="c", subcore_axis_name="s",
                                num_cores=_NC, num_subcores=_NS),
    compiler_params=pltpu.CompilerParams(needs_layout_passes=False),
    scratch_types=[
        pltpu.VMEM((_B,), jnp.float32),
        pltpu.VMEM((_B,), jnp.float32),
        pltpu.VMEM((_B,), jnp.float32),
        pltpu.VMEM((_B,), jnp.float32),
        pltpu.VMEM((_B,), jnp.float32),
        pltpu.VMEM((_B,), jnp.float32),
        pltpu.VMEM((_B,), jnp.int32),
        pltpu.VMEM((_B,), jnp.int32),
        pltpu.VMEM((_B,), jnp.int32),
        pltpu.VMEM((_B,), jnp.float32),
        pltpu.VMEM((_B,), jnp.float32),
        pltpu.VMEM((_B,), jnp.float32),
        pltpu.VMEM((16,), jnp.int32),
        pltpu.VMEM((_NSEG_PAD,), jnp.float32),
        pltpu.VMEM_SHARED((_NSEG_PAD,), jnp.float32),
        pltpu.SemaphoreType.DMA,
        pltpu.SemaphoreType.DMA,
        pltpu.SemaphoreType.DMA,
        pltpu.SemaphoreType.DMA,
        pltpu.SemaphoreType.DMA,
        pltpu.SemaphoreType.DMA,
        pltpu.SemaphoreType.DMA,
    ],
)


def _combine_body(p_ref, o_ref):
    o_ref[...] = p_ref[0] + p_ref[1]


def kernel(Ea, Eza, N):
    total, part = _sc_call(Ea, Eza, N)
    comb = pl.pallas_call(
        _combine_body,
        out_shape=jax.ShapeDtypeStruct((_NSEG_PAD // 128, 128), jnp.float32),
    )(part.reshape(_NC, _NSEG_PAD // 128, 128))
    E = comb.reshape(-1)[:_NSEG]
    return (total, E)


# speedup vs baseline: 1.2369x; 1.2014x over previous
import jax
import jax.numpy as jnp
from jax import lax
from jax.experimental import pallas as pl
from jax.experimental.pallas import tpu as pltpu
from jax.experimental.pallas import tpu_sc as plsc

_N_ELEMS = 6400000
_NSEG = 100000
_NC, _NS = 2, 16
_NW = _NC * _NS
_NSEG_PAD = 100096
_PER_TILE = _N_ELEMS // _NW
_B = 4000
_NCHUNK = _PER_TILE // _B
_NPAIR = _NCHUNK // 2
_VREGS = _B // 16
_UNROLL = 5
_FLUSH = _NSEG_PAD // _NS


def _sc_body(ea_hbm, eza_hbm, n_hbm, out_hbm, part_hbm,
             ea0, ea1, ez0, ez1, n0, n1, nb, accl, accs,
             isem0, isem1, osem0, osem1):
    c = lax.axis_index("c")
    s = lax.axis_index("s")
    wid = c * _NS + s
    base_el = wid * _PER_TILE

    ea = (ea0, ea1)
    ez = (ez0, ez1)
    nn = (n0, n1)
    tt = ea
    isem = (isem0, isem1)
    osem = (osem0, osem1)

    zero16 = jnp.zeros((16,), jnp.float32)

    def z16(i, _):
        for u in range(16):
            accl[pl.ds((i * 16 + u) * 16, 16)] = zero16
        return 0
    lax.fori_loop(0, _NSEG_PAD // 256, z16, 0)

    pltpu.sync_copy(accl.at[pl.ds(0, _FLUSH)],
                    accs.at[pl.ds(s * _FLUSH, _FLUSH)])
    plsc.subcore_barrier()

    def start_in(slot, k):
        off = base_el + k * _B
        pltpu.async_copy(ea_hbm.at[pl.ds(off, _B)], ea[slot], isem[slot])
        pltpu.async_copy(eza_hbm.at[pl.ds(off, _B)], ez[slot], isem[slot])
        pltpu.async_copy(n_hbm.at[pl.ds(off, _B)], nn[slot], isem[slot])

    def wait_in(slot, k):
        off = base_el + k * _B
        pltpu.make_async_copy(ea_hbm.at[pl.ds(off, _B)], ea[slot], isem[slot]).wait()
        pltpu.make_async_copy(eza_hbm.at[pl.ds(off, _B)], ez[slot], isem[slot]).wait()
        pltpu.make_async_copy(n_hbm.at[pl.ds(off, _B)], nn[slot], isem[slot]).wait()

    def wait_out(slot, k):
        off = base_el + k * _B
        pltpu.make_async_copy(tt[slot], out_hbm.at[pl.ds(off, _B)], osem[slot]).wait()

    iota16 = lax.iota(jnp.int32, 16)
    stride_idx = iota16 * _VREGS

    def compute(slot, k):
        @plsc.parallel_loop(0, _VREGS, unroll=_UNROLL)
        def _(i):
            sl = pl.ds(i * 16, 16)
            ea[slot][sl] = ea[slot][sl] + ez[slot][sl]
        pltpu.async_copy(tt[slot], out_hbm.at[pl.ds(base_el + k * _B, _B)],
                         osem[slot])

        @plsc.parallel_loop(0, _VREGS, unroll=_UNROLL)
        def _(i):
            idx = stride_idx + i
            t = plsc.load_gather(tt[slot], [idx])
            n16 = plsc.load_gather(nn[slot], [idx])
            plsc.addupdate_scatter(accl, [n16], t)

    start_in(0, 0)
    start_in(1, 1)

    def pair(k2, _):
        k = 2 * k2
        wait_in(0, k)
        compute(0, k)
        wait_out(0, k)

        @pl.when(k2 < _NPAIR - 1)
        def _():
            start_in(0, k + 2)
        wait_in(1, k + 1)
        compute(1, k + 1)
        wait_out(1, k + 1)

        @pl.when(k2 < _NPAIR - 1)
        def _():
            start_in(1, k + 3)
        return 0
    lax.fori_loop(0, _NPAIR, pair, 0)

    pltpu.sync_copy(n_hbm.at[pl.ds(base_el, 16)], nb)
    lo = nb[pl.ds(0, 16)][0]
    pltpu.sync_copy(n_hbm.at[pl.ds(base_el + _PER_TILE - 16, 16)], nb)
    hi = nb[pl.ds(0, 16)][15]

    fbase = (lo // 256) * 256
    nbat = (hi + 1 - fbase + 255) // 256

    def fl(b, _):
        off_b = fbase + b * 256
        for u in range(16):
            off = off_b + u * 16
            pltpu.async_copy(accl.at[pl.ds(off, 16)], accs.at[off + iota16],
                             isem0, add=True)
        for u in range(16):
            off = off_b + u * 16
            pltpu.make_async_copy(accl.at[pl.ds(off, 16)],
                                  accs.at[off + iota16], isem0).wait()
        return 0
    lax.fori_loop(0, nbat, fl, 0)

    plsc.subcore_barrier()
    off = s * _FLUSH
    pltpu.sync_copy(accs.at[pl.ds(off, _FLUSH)], accl.at[pl.ds(0, _FLUSH)])
    pltpu.sync_copy(accl.at[pl.ds(0, _FLUSH)],
                    part_hbm.at[pl.ds(c * _NSEG_PAD + off, _FLUSH)])


_sc_call = pl.kernel(
    _sc_body,
    out_type=(
        jax.ShapeDtypeStruct((_N_ELEMS,), jnp.float32),
        jax.ShapeDtypeStruct((_NC * _NSEG_PAD,), jnp.float32),
    ),
    mesh=plsc.VectorSubcoreMesh(core_axis_name="c", subcore_axis_name="s",
                                num_cores=_NC, num_subcores=_NS),
    compiler_params=pltpu.CompilerParams(needs_layout_passes=False),
    scratch_types=[
        pltpu.VMEM((_B,), jnp.float32),
        pltpu.VMEM((_B,), jnp.float32),
        pltpu.VMEM((_B,), jnp.float32),
        pltpu.VMEM((_B,), jnp.float32),
        pltpu.VMEM((_B,), jnp.int32),
        pltpu.VMEM((_B,), jnp.int32),
        pltpu.VMEM((16,), jnp.int32),
        pltpu.VMEM((_NSEG_PAD,), jnp.float32),
        pltpu.VMEM_SHARED((_NSEG_PAD,), jnp.float32),
        pltpu.SemaphoreType.DMA,
        pltpu.SemaphoreType.DMA,
        pltpu.SemaphoreType.DMA,
        pltpu.SemaphoreType.DMA,
    ],
)


def _combine_body(p_ref, o_ref):
    o_ref[...] = p_ref[0] + p_ref[1]


def kernel(Ea, Eza, N):
    total, part = _sc_call(Ea, Eza, N)
    comb = pl.pallas_call(
        _combine_body,
        out_shape=jax.ShapeDtypeStruct((_NSEG_PAD // 128, 128), jnp.float32),
    )(part.reshape(_NC, _NSEG_PAD // 128, 128))
    E = comb.reshape(-1)[:_NSEG]
    return (total, E)
